# Initial kernel scaffold; baseline (speedup 1.0000x reference)
import jax
import jax.numpy as jnp
from jax.experimental import pallas as pl


def kernel(input, target):
    raise NotImplementedError("write your pallas kernel here")


# speedup vs baseline: 10.6040x; 10.6040x over previous
import functools

import jax
import jax.numpy as jnp
from jax.experimental import pallas as pl
from jax.experimental.pallas import tpu as pltpu

B, C, H, W = 8, 21, 384, 384
N = H * W
K = N // 2

BH = 64


def _ce_kernel(x_ref, t_ref, nll_ref):
    x = x_ref[0]
    t = t_ref[0]
    m = jnp.max(x, axis=0)
    s = jnp.sum(jnp.exp(x - m[None]), axis=0)
    cls = jax.lax.broadcasted_iota(jnp.int32, (C, BH, W), 0)
    tl = jnp.sum(jnp.where(cls == t[None], x, 0.0), axis=0)
    nll_ref[0] = (m - tl) + jnp.log(s)


def _key_of(bits):
    mask = jnp.int32(0x7FFFFFFF)
    return jnp.where(bits >= 0, bits, bits ^ mask)


def _select_kernel(nll_ref, acc_ref):
    b = pl.program_id(0)
    v = nll_ref[0]
    bits = jax.lax.bitcast_convert_type(v, jnp.int32)
    key = _key_of(bits)

    def body(_, lohi):
        lo, hi = lohi
        mid = (lo >> 1) + (hi >> 1) + (lo & hi & 1)
        cnt = jnp.sum((key > mid).astype(jnp.int32))
        go_low = cnt < K
        return (jnp.where(go_low, lo, mid), jnp.where(go_low, mid, hi))

    lo0 = jnp.int32(-2147483647 - 1)
    hi0 = jnp.int32(2147483647)
    _, t_star = jax.lax.fori_loop(0, 32, body, (lo0, hi0))

    gt = key > t_star
    cnt_gt = jnp.sum(gt.astype(jnp.int32))
    sum_gt = jnp.sum(jnp.where(gt, v, 0.0))
    tbits = jnp.where(t_star >= 0, t_star, t_star ^ jnp.int32(0x7FFFFFFF))
    tval = jax.lax.bitcast_convert_type(tbits, jnp.float32)
    topk_sum = sum_gt + (K - cnt_gt).astype(jnp.float32) * tval

    prev = jnp.where(b == 0, jnp.zeros((1, 1), jnp.float32), acc_ref[...])
    acc_ref[...] = prev + topk_sum


@jax.jit
def kernel(input, target):
    target = target.astype(jnp.int32)

    nll = pl.pallas_call(
        _ce_kernel,
        grid=(B, H // BH),
        in_specs=[
            pl.BlockSpec((1, C, BH, W), lambda b, h: (b, 0, h, 0)),
            pl.BlockSpec((1, BH, W), lambda b, h: (b, h, 0)),
        ],
        out_specs=pl.BlockSpec((1, BH, W), lambda b, h: (b, h, 0)),
        out_shape=jax.ShapeDtypeStruct((B, H, W), jnp.float32),
    )(input, target)

    acc = pl.pallas_call(
        _select_kernel,
        grid=(B,),
        in_specs=[pl.BlockSpec((1, H, W), lambda b: (b, 0, 0))],
        out_specs=pl.BlockSpec((1, 1), lambda b: (0, 0)),
        out_shape=jax.ShapeDtypeStruct((1, 1), jnp.float32),
    )(nll)

    return acc[0, 0] / (B * K)


# speedup vs baseline: 16.8375x; 1.5878x over previous
import jax
import jax.numpy as jnp
from jax.experimental import pallas as pl

B, C, H, W = 8, 21, 384, 384
N = H * W
K = N // 2

BH = 64


def _ce_kernel(x_ref, t_ref, nll_ref):
    x = x_ref[0]
    t = t_ref[0]
    m = jnp.max(x, axis=0)
    s = jnp.sum(jnp.exp(x - m[None]), axis=0)
    cls = jax.lax.broadcasted_iota(jnp.int32, (C, BH, W), 0)
    tl = jnp.sum(jnp.where(cls == t[None], x, 0.0), axis=0)
    nll_ref[0] = (m - tl) + jnp.log(s)


def _select_kernel(nll_ref, acc_ref):
    v = nll_ref[...]
    bits = jax.lax.bitcast_convert_type(v, jnp.int32)
    mask = jnp.int32(0x7FFFFFFF)
    key = jnp.where(bits >= 0, bits, bits ^ mask)

    def body(_, lohi):
        lo, hi = lohi
        mid = (lo >> 1) + (hi >> 1) + (lo & hi & 1)
        cnt = jnp.sum((key > mid).astype(jnp.int32), axis=(1, 2), keepdims=True)
        go_low = cnt < K
        return (jnp.where(go_low, lo, mid), jnp.where(go_low, mid, hi))

    lo0 = jnp.full((B, 1, 1), -2147483647 - 1, jnp.int32)
    hi0 = jnp.full((B, 1, 1), 2147483647, jnp.int32)
    _, t_star = jax.lax.fori_loop(0, 32, body, (lo0, hi0))

    gt = key > t_star
    cnt_gt = jnp.sum(gt.astype(jnp.int32), axis=(1, 2), keepdims=True)
    sum_gt = jnp.sum(jnp.where(gt, v, 0.0), axis=(1, 2), keepdims=True)
    tbits = jnp.where(t_star >= 0, t_star, t_star ^ mask)
    tval = jax.lax.bitcast_convert_type(tbits, jnp.float32)
    topk = sum_gt + (K - cnt_gt).astype(jnp.float32) * tval
    acc_ref[...] = jnp.sum(topk, axis=0)


@jax.jit
def kernel(input, target):
    target = target.astype(jnp.int32)

    nll = pl.pallas_call(
        _ce_kernel,
        grid=(B, H // BH),
        in_specs=[
            pl.BlockSpec((1, C, BH, W), lambda b, h: (b, 0, h, 0)),
            pl.BlockSpec((1, BH, W), lambda b, h: (b, h, 0)),
        ],
        out_specs=pl.BlockSpec((1, BH, W), lambda b, h: (b, h, 0)),
        out_shape=jax.ShapeDtypeStruct((B, H, W), jnp.float32),
    )(input, target)

    acc = pl.pallas_call(
        _select_kernel,
        out_shape=jax.ShapeDtypeStruct((1, 1), jnp.float32),
    )(nll)

    return acc[0, 0] / (B * K)
